# Initial kernel scaffold; baseline (speedup 1.0000x reference)
import jax
import jax.numpy as jnp
from jax.experimental import pallas as pl


def kernel(x, edge_index, batch, W1, b1, W2, b2, Wc, bc):
    raise NotImplementedError("write your pallas kernel here")


# speedup vs baseline: 12.3595x; 12.3595x over previous
import functools

import jax
import jax.numpy as jnp
from jax import lax
from jax.experimental import pallas as pl
from jax.experimental.pallas import tpu as pltpu
from jax.experimental.pallas import tpu_sc as plsc

_N = 10000
_E = 320000
_D = 128
_G = 64
_C = 10

_NC = 2
_NS = 16
_NW = _NC * _NS
_EPW = _E // _NW
_EK = 80
_ESTEPS = _EPW // _EK
_ZR = 632
_ZR_LAST = _N - 15 * _ZR
_DEGW = 128

_ROWBLK = 1000


def _gelu_exact(x):
    return 0.5 * x * (1.0 + lax.erf(x * 0.7071067811865476))


def _sc_degree(dst, ones_blk, zeros_deg):
    mesh = plsc.VectorSubcoreMesh(core_axis_name="c", subcore_axis_name="s")

    @functools.partial(
        pl.kernel,
        out_type=jax.ShapeDtypeStruct((_NC, _N, _DEGW), jnp.float32),
        mesh=mesh,
        scratch_types=[
            pltpu.VMEM((_EK,), jnp.int32),
            pltpu.VMEM((_EK, _DEGW), jnp.float32),
            pltpu.VMEM_SHARED((_N, _DEGW), jnp.float32),
        ],
    )
    def k(dst_hbm, ones_hbm, zeros_hbm, out_hbm, didx, ones_v, acc):
        c = lax.axis_index("c")
        s = lax.axis_index("s")
        wid = c * _NS + s

        @pl.when(s < _NS - 1)
        def _():
            pltpu.sync_copy(zeros_hbm.at[pl.ds(0, _ZR)],
                            acc.at[pl.ds(s * _ZR, _ZR)])

        @pl.when(s == _NS - 1)
        def _():
            pltpu.sync_copy(zeros_hbm.at[pl.ds(0, _ZR_LAST)],
                            acc.at[pl.ds(15 * _ZR, _ZR_LAST)])

        pltpu.sync_copy(ones_hbm, ones_v)
        plsc.subcore_barrier()
        ebase = wid * _EPW

        @pl.loop(0, _ESTEPS)
        def _(i):
            pltpu.sync_copy(dst_hbm.at[pl.ds(ebase + i * _EK, _EK)], didx)
            pltpu.sync_copy(ones_v, acc.at[didx], add=True)

        plsc.subcore_barrier()

        @pl.when(s < _NS - 1)
        def _():
            pltpu.sync_copy(acc.at[pl.ds(s * _ZR, _ZR)],
                            out_hbm.at[c, pl.ds(s * _ZR, _ZR)])

        @pl.when(s == _NS - 1)
        def _():
            pltpu.sync_copy(acc.at[pl.ds(15 * _ZR, _ZR_LAST)],
                            out_hbm.at[c, pl.ds(15 * _ZR, _ZR_LAST)])

    return k(dst, ones_blk, zeros_deg)


def _sc_scatter(hp, src, dst, zeros_blk):
    mesh = plsc.VectorSubcoreMesh(core_axis_name="c", subcore_axis_name="s")

    @functools.partial(
        pl.kernel,
        out_type=jax.ShapeDtypeStruct((_NC, _N, _D), jnp.float32),
        mesh=mesh,
        scratch_types=[
            pltpu.VMEM((_EK,), jnp.int32),
            pltpu.VMEM((_EK,), jnp.int32),
            pltpu.VMEM((_EK, _D), jnp.float32),
            pltpu.VMEM_SHARED((_N, _D), jnp.float32),
        ],
    )
    def k(hp_hbm, src_hbm, dst_hbm, zeros_hbm, out_hbm, sidx, didx, rows, acc):
        c = lax.axis_index("c")
        s = lax.axis_index("s")
        wid = c * _NS + s

        @pl.when(s < _NS - 1)
        def _():
            pltpu.sync_copy(zeros_hbm.at[pl.ds(0, _ZR)],
                            acc.at[pl.ds(s * _ZR, _ZR)])

        @pl.when(s == _NS - 1)
        def _():
            pltpu.sync_copy(zeros_hbm.at[pl.ds(0, _ZR_LAST)],
                            acc.at[pl.ds(15 * _ZR, _ZR_LAST)])

        plsc.subcore_barrier()
        ebase = wid * _EPW

        @pl.loop(0, _ESTEPS)
        def _(i):
            pltpu.sync_copy(src_hbm.at[pl.ds(ebase + i * _EK, _EK)], sidx)
            pltpu.sync_copy(dst_hbm.at[pl.ds(ebase + i * _EK, _EK)], didx)
            pltpu.sync_copy(hp_hbm.at[sidx], rows)
            pltpu.sync_copy(rows, acc.at[didx], add=True)

        plsc.subcore_barrier()

        @pl.when(s < _NS - 1)
        def _():
            pltpu.sync_copy(acc.at[pl.ds(s * _ZR, _ZR)],
                            out_hbm.at[c, pl.ds(s * _ZR, _ZR)])

        @pl.when(s == _NS - 1)
        def _():
            pltpu.sync_copy(acc.at[pl.ds(15 * _ZR, _ZR_LAST)],
                            out_hbm.at[c, pl.ds(15 * _ZR, _ZR_LAST)])

    return k(hp, src, dst, zeros_blk)


def _tc_prep1(degp, x, W1):
    def body(degp_ref, x_ref, w_ref, hp_ref, inv_ref):
        dp = degp_ref[0] + degp_ref[1]
        inv = lax.rsqrt(dp[:, 0:1] + 1.0)
        h = jnp.dot(x_ref[...], w_ref[...], preferred_element_type=jnp.float32)
        hp_ref[...] = h * inv
        inv_ref[...] = inv

    return pl.pallas_call(
        body,
        grid=(_N // _ROWBLK,),
        in_specs=[
            pl.BlockSpec((_NC, _ROWBLK, _DEGW), lambda i: (0, i, 0)),
            pl.BlockSpec((_ROWBLK, _D), lambda i: (i, 0)),
            pl.BlockSpec((_D, _D), lambda i: (0, 0)),
        ],
        out_specs=[
            pl.BlockSpec((_ROWBLK, _D), lambda i: (i, 0)),
            pl.BlockSpec((_ROWBLK, 1), lambda i: (i, 0)),
        ],
        out_shape=[
            jax.ShapeDtypeStruct((_N, _D), jnp.float32),
            jax.ShapeDtypeStruct((_N, 1), jnp.float32),
        ],
    )(degp, x, W1)


def _tc_combine_prep(part, hp, inv, b, W):
    def body(part_ref, hp_ref, inv_ref, b_ref, w_ref, out_ref):
        inv_v = inv_ref[...]
        t = (part_ref[0] + part_ref[1] + hp_ref[...]) * inv_v + b_ref[...]
        h = _gelu_exact(t)
        out_ref[...] = jnp.dot(h, w_ref[...],
                               preferred_element_type=jnp.float32) * inv_v

    return pl.pallas_call(
        body,
        grid=(_N // _ROWBLK,),
        in_specs=[
            pl.BlockSpec((_NC, _ROWBLK, _D), lambda i: (0, i, 0)),
            pl.BlockSpec((_ROWBLK, _D), lambda i: (i, 0)),
            pl.BlockSpec((_ROWBLK, 1), lambda i: (i, 0)),
            pl.BlockSpec((1, _D), lambda i: (0, 0)),
            pl.BlockSpec((_D, _D), lambda i: (0, 0)),
        ],
        out_specs=pl.BlockSpec((_ROWBLK, _D), lambda i: (i, 0)),
        out_shape=jax.ShapeDtypeStruct((_N, _D), jnp.float32),
    )(part, hp, inv, b, W)


def _tc_final(part, hp, inv, b, batch_row, Wc, bc):
    steps = _N // _ROWBLK

    def body(part_ref, hp_ref, inv_ref, b_ref, bat_ref, wc_ref, bc_ref,
             out_ref, accs, accc):
        i = pl.program_id(0)

        @pl.when(i == 0)
        def _():
            accs[...] = jnp.zeros_like(accs)
            accc[...] = jnp.zeros_like(accc)

        inv_v = inv_ref[...]
        t = (part_ref[0] + part_ref[1] + hp_ref[...]) * inv_v + b_ref[...]
        h = _gelu_exact(t)
        oht = (bat_ref[0] ==
               lax.broadcasted_iota(jnp.int32, (_G, _ROWBLK), 0)
               ).astype(jnp.float32)
        accs[...] += jnp.dot(oht, h, preferred_element_type=jnp.float32)
        accc[...] += jnp.dot(oht, jnp.ones((_ROWBLK, 1), jnp.float32),
                             preferred_element_type=jnp.float32)

        @pl.when(i == steps - 1)
        def _():
            g = accs[...] / jnp.maximum(accc[...], 1.0)
            out_ref[...] = jnp.dot(g, wc_ref[...],
                                   preferred_element_type=jnp.float32) + bc_ref[...]

    return pl.pallas_call(
        body,
        grid=(steps,),
        in_specs=[
            pl.BlockSpec((_NC, _ROWBLK, _D), lambda i: (0, i, 0)),
            pl.BlockSpec((_ROWBLK, _D), lambda i: (i, 0)),
            pl.BlockSpec((_ROWBLK, 1), lambda i: (i, 0)),
            pl.BlockSpec((1, _D), lambda i: (0, 0)),
            pl.BlockSpec((1, 1, _ROWBLK), lambda i: (i, 0, 0)),
            pl.BlockSpec((_D, _C), lambda i: (0, 0)),
            pl.BlockSpec((1, _C), lambda i: (0, 0)),
        ],
        out_specs=pl.BlockSpec((_G, _C), lambda i: (0, 0)),
        out_shape=jax.ShapeDtypeStruct((_G, _C), jnp.float32),
        scratch_shapes=[
            pltpu.VMEM((_G, _D), jnp.float32),
            pltpu.VMEM((_G, 1), jnp.float32),
        ],
    )(part, hp, inv, b, batch_row, Wc, bc)


def kernel(x, edge_index, batch, W1, b1, W2, b2, Wc, bc):
    src = edge_index[0]
    dst = edge_index[1]
    ones_blk = jnp.ones((_EK, _DEGW), jnp.float32)
    zeros_blk = jnp.zeros((_ZR, _D), jnp.float32)
    zeros_deg = zeros_blk

    degp = _sc_degree(dst, ones_blk, zeros_deg)
    hp1, inv = _tc_prep1(degp, x, W1)
    part1 = _sc_scatter(hp1, src, dst, zeros_blk)
    hp2 = _tc_combine_prep(part1, hp1, inv, b1.reshape(1, _D), W2)
    part2 = _sc_scatter(hp2, src, dst, zeros_blk)
    return _tc_final(part2, hp2, inv, b2.reshape(1, _D),
                     batch.reshape(_N // _ROWBLK, 1, _ROWBLK), Wc,
                     bc.reshape(1, _C))
